# baseline (speedup 1.0000x reference)
import functools

import jax
import jax.numpy as jnp
import numpy as np
from jax import lax
from jax.experimental import pallas as pl
from jax.experimental.pallas import tpu as pltpu
from jax.experimental.pallas import tpu_sc as plsc

N = 4096
K_CAP = 48
COORD_DIM = 3
NUM_FREQ = 16
EMB = COORD_DIM * NUM_FREQ * 2
RADIUS2 = 0.1 * 0.1
CH = 64
NE = N * K_CAP

def _gelu(v):
    return v * 0.5 * (1.0 + lax.erf(v * np.float32(1.0 / np.sqrt(2.0))))


def _embed(t):
    fiota = lax.broadcasted_iota(jnp.int32, (1, NUM_FREQ), 1).astype(jnp.float32)
    freqs = jnp.exp(fiota * np.float32(np.log(1.0 / 10000.0) / NUM_FREQ))
    parts = [lax.slice(t, (0, c), (t.shape[0], c + 1)) * freqs
             for c in range(COORD_DIM)]
    prod = jnp.concatenate(parts, axis=1)
    return jnp.concatenate([jnp.sin(prod), jnp.cos(prod)], axis=1)


def _prep_body(y_ref, x_ref, f_ref, w0_ref, b0_ref, g_ref, b_ref):
    w0a = w0_ref[:EMB, :]
    w0b = w0_ref[EMB:, :]
    a = jnp.dot(_embed(y_ref[...]), w0a, preferred_element_type=jnp.float32)
    g_ref[...] = jnp.concatenate([a, f_ref[...]], axis=1)
    b_ref[...] = (jnp.dot(_embed(x_ref[...]), w0b,
                          preferred_element_type=jnp.float32)
                  + b0_ref[...])


def _prep(y, x, f_y, W0, b0):
    return pl.pallas_call(
        _prep_body,
        out_shape=(jax.ShapeDtypeStruct((N, 2 * CH), jnp.float32),
                   jax.ShapeDtypeStruct((N, CH), jnp.float32)),
    )(y, x, f_y, W0, b0.reshape(1, CH))


_NC = 2
_NS = 16
_NW = _NC * _NS
_GCH = 128


_NBUF = 3


def _gather_edges(G, flat_idx):
    per_w = NE // _NW
    nch = per_w // _GCH
    mesh = plsc.VectorSubcoreMesh(core_axis_name="c", subcore_axis_name="s")

    @functools.partial(
        pl.kernel, mesh=mesh,
        out_type=jax.ShapeDtypeStruct((NE, 2 * CH), jnp.float32),
        scratch_types=(
            [pltpu.VMEM((_GCH,), jnp.int32)] * _NBUF
            + [pltpu.VMEM((_GCH, 2 * CH), jnp.float32)] * _NBUF
            + [pltpu.SemaphoreType.DMA] * (2 * _NBUF)
        ),
    )
    def gather_k(g_hbm, idx_hbm, out_hbm, *bufs):
        idxs = bufs[:_NBUF]
        rows = bufs[_NBUF:2 * _NBUF]
        gsem = bufs[2 * _NBUF:3 * _NBUF]
        wsem = bufs[3 * _NBUF:4 * _NBUF]
        wid = lax.axis_index("s") * _NC + lax.axis_index("c")
        base = wid * per_w

        gcp = [None] * nch
        wcp = [None] * nch
        for ci in range(nch):
            b = ci % _NBUF
            if ci >= _NBUF:
                wcp[ci - _NBUF].wait()
            off = base + ci * _GCH
            pltpu.sync_copy(idx_hbm.at[pl.ds(off, _GCH)], idxs[b])
            gcp[ci] = pltpu.async_copy(g_hbm.at[idxs[b]], rows[b], gsem[b])
            if ci >= _NBUF - 1:
                j = ci - (_NBUF - 1)
                bj = j % _NBUF
                gcp[j].wait()
                wcp[j] = pltpu.async_copy(
                    rows[bj], out_hbm.at[pl.ds(base + j * _GCH, _GCH)],
                    wsem[bj])
        for j in range(nch - (_NBUF - 1), nch):
            b = j % _NBUF
            gcp[j].wait()
            wcp[j] = pltpu.async_copy(
                rows[b], out_hbm.at[pl.ds(base + j * _GCH, _GCH)], wsem[b])
        for j in range(nch - _NBUF, nch):
            wcp[j].wait()

    return gather_k(G, flat_idx)


_QW = N // _NW
_NG = _QW // 16
_UNROLL = 8


def _search(x0, x1, x2, y0, y1, y2):
    mesh = plsc.VectorSubcoreMesh(core_axis_name="c", subcore_axis_name="s")

    @functools.partial(
        pl.kernel, mesh=mesh,
        compiler_params=pltpu.CompilerParams(needs_layout_passes=False),
        out_type=(jax.ShapeDtypeStruct((NE,), jnp.int32),
                  jax.ShapeDtypeStruct((N,), jnp.int32)),
        scratch_types=[
            pltpu.VMEM((N,), jnp.float32),
            pltpu.VMEM((N,), jnp.float32),
            pltpu.VMEM((N,), jnp.float32),
            pltpu.VMEM((_QW,), jnp.float32),
            pltpu.VMEM((_QW,), jnp.float32),
            pltpu.VMEM((_QW,), jnp.float32),
            pltpu.VMEM((_QW * K_CAP,), jnp.int32),
            pltpu.VMEM((_QW,), jnp.int32),
        ],
    )
    def search_k(x0h, x1h, x2h, y0h, y1h, y2h, idxh, cnth,
                 y0v, y1v, y2v, x0v, x1v, x2v, oidx, ocnt):
        wid = lax.axis_index("s") * _NC + lax.axis_index("c")
        qbase = wid * _QW
        pltpu.sync_copy(y0h, y0v)
        pltpu.sync_copy(y1h, y1v)
        pltpu.sync_copy(y2h, y2v)
        pltpu.sync_copy(x0h.at[pl.ds(qbase, _QW)], x0v)
        pltpu.sync_copy(x1h.at[pl.ds(qbase, _QW)], x1v)
        pltpu.sync_copy(x2h.at[pl.ds(qbase, _QW)], x2v)

        lanes = lax.broadcasted_iota(jnp.int32, (16,), 0)
        zeros16 = jnp.zeros((16,), jnp.int32)

        def fill(z, carry):
            oidx[pl.ds(z * 16, 16)] = (lanes + z * 16) & (N - 1)
            return carry
        lax.fori_loop(0, _QW * K_CAP // 16, fill, 0)

        def per_group(g, carry):
            xg0 = x0v[pl.ds(g * 16, 16)]
            xg1 = x1v[pl.ds(g * 16, 16)]
            xg2 = x2v[pl.ds(g * 16, 16)]
            rowb = (g * 16 + lanes) * K_CAP

            @plsc.parallel_loop(0, N, step=_UNROLL, carry=zeros16)
            def per_cand(jj, cursor):
                for u in range(_UNROLL):
                    jsplat = zeros16 + (jj + u)
                    d0 = plsc.load_gather(y0v, [jsplat]) - xg0
                    d1 = plsc.load_gather(y1v, [jsplat]) - xg1
                    d2 = plsc.load_gather(y2v, [jsplat]) - xg2
                    dist = d0 * d0 + d1 * d1 + d2 * d2
                    ok = jnp.logical_and(dist <= RADIUS2, cursor < K_CAP)
                    plsc.store_scatter(oidx, [rowb + cursor], jsplat, mask=ok)
                    cursor = cursor + ok.astype(jnp.int32)
                return cursor

            cursor = per_cand
            ocnt[pl.ds(g * 16, 16)] = cursor
            return carry

        lax.fori_loop(0, _NG, per_group, 0)

        pltpu.sync_copy(oidx, idxh.at[pl.ds(qbase * K_CAP, _QW * K_CAP)])
        pltpu.sync_copy(ocnt, cnth.at[pl.ds(qbase, _QW)])

    return search_k(x0, x1, x2, y0, y1, y2)


_BQ = 64


def _mlp_body(e_ref, b_ref, cnt_ref, w1_ref, b1_ref, w2_ref, b2_ref, o_ref):
    e = e_ref[...]
    ea = lax.slice(e, (0, 0), (_BQ * K_CAP, CH))
    ef = lax.slice(e, (0, CH), (_BQ * K_CAP, 2 * CH))
    t = ea.reshape(_BQ, K_CAP, CH) + b_ref[...][:, None, :]
    t = _gelu(t).reshape(_BQ * K_CAP, CH)
    t = _gelu(jnp.dot(t, w1_ref[...], preferred_element_type=jnp.float32)
              + b1_ref[...])
    v = jnp.dot(t, w2_ref[...], preferred_element_type=jnp.float32) + b2_ref[...]
    v = v * ef
    slot = lax.broadcasted_iota(jnp.int32, (_BQ, K_CAP), 1)
    mask = (slot < cnt_ref[...]).astype(jnp.float32)
    v = v.reshape(_BQ, K_CAP, CH) * mask[:, :, None]
    o_ref[...] = jnp.sum(v, axis=1)


def _mlp(E, B, cnt, W1, b1, W2, b2):
    grid = N // _BQ
    return pl.pallas_call(
        _mlp_body,
        grid=(grid,),
        in_specs=[
            pl.BlockSpec((_BQ * K_CAP, 2 * CH), lambda i: (i, 0)),
            pl.BlockSpec((_BQ, CH), lambda i: (i, 0)),
            pl.BlockSpec((_BQ, 1), lambda i: (i, 0)),
            pl.BlockSpec((CH, CH), lambda i: (0, 0)),
            pl.BlockSpec((1, CH), lambda i: (0, 0)),
            pl.BlockSpec((CH, CH), lambda i: (0, 0)),
            pl.BlockSpec((1, CH), lambda i: (0, 0)),
        ],
        out_specs=pl.BlockSpec((_BQ, CH), lambda i: (i, 0)),
        out_shape=jax.ShapeDtypeStruct((N, CH), jnp.float32),
    )(E, B, cnt.reshape(N, 1), W1, b1.reshape(1, CH), W2, b2.reshape(1, CH))


def kernel(y, x, f_y, W0, b0, W1, b1, W2, b2):
    idx_flat, cnt = _search(x[:, 0], x[:, 1], x[:, 2],
                            y[:, 0], y[:, 1], y[:, 2])
    G, B = _prep(y, x, f_y, W0, b0)
    E = _gather_edges(G, idx_flat)
    return _mlp(E, B, cnt, W1, b1, W2, b2)


# speedup vs baseline: 1.5989x; 1.5989x over previous
import functools

import jax
import jax.numpy as jnp
import numpy as np
from jax import lax
from jax.experimental import pallas as pl
from jax.experimental.pallas import tpu as pltpu
from jax.experimental.pallas import tpu_sc as plsc

N = 4096
K_CAP = 48
COORD_DIM = 3
NUM_FREQ = 16
EMB = COORD_DIM * NUM_FREQ * 2
RADIUS2 = 0.1 * 0.1
CH = 64
NE = N * K_CAP

def _gelu(v):
    return v * 0.5 * (1.0 + lax.erf(v * np.float32(1.0 / np.sqrt(2.0))))


def _embed(t):
    fiota = lax.broadcasted_iota(jnp.int32, (1, NUM_FREQ), 1).astype(jnp.float32)
    freqs = jnp.exp(fiota * np.float32(np.log(1.0 / 10000.0) / NUM_FREQ))
    parts = [lax.slice(t, (0, c), (t.shape[0], c + 1)) * freqs
             for c in range(COORD_DIM)]
    prod = jnp.concatenate(parts, axis=1)
    return jnp.concatenate([jnp.sin(prod), jnp.cos(prod)], axis=1)


def _prep_body(y_ref, x_ref, f_ref, w0_ref, b0_ref, g_ref, b_ref):
    w0a = w0_ref[:EMB, :]
    w0b = w0_ref[EMB:, :]
    a = jnp.dot(_embed(y_ref[...]), w0a, preferred_element_type=jnp.float32)
    g_ref[...] = jnp.concatenate([a, f_ref[...]], axis=1)
    b_ref[...] = (jnp.dot(_embed(x_ref[...]), w0b,
                          preferred_element_type=jnp.float32)
                  + b0_ref[...])


def _prep(y, x, f_y, W0, b0):
    return pl.pallas_call(
        _prep_body,
        out_shape=(jax.ShapeDtypeStruct((N, 2 * CH), jnp.float32),
                   jax.ShapeDtypeStruct((N, CH), jnp.float32)),
    )(y, x, f_y, W0, b0.reshape(1, CH))


_NC = 2
_NS = 16
_NW = _NC * _NS
_GCH = 128


_NBUF = 3


def _gather_edges(G, flat_idx):
    per_w = NE // _NW
    nch = per_w // _GCH
    mesh = plsc.VectorSubcoreMesh(core_axis_name="c", subcore_axis_name="s")

    @functools.partial(
        pl.kernel, mesh=mesh,
        out_type=jax.ShapeDtypeStruct((NE, 2 * CH), jnp.float32),
        scratch_types=(
            [pltpu.VMEM((_GCH,), jnp.int32)] * _NBUF
            + [pltpu.VMEM((_GCH, 2 * CH), jnp.float32)] * _NBUF
            + [pltpu.SemaphoreType.DMA] * (2 * _NBUF)
        ),
    )
    def gather_k(g_hbm, idx_hbm, out_hbm, *bufs):
        idxs = bufs[:_NBUF]
        rows = bufs[_NBUF:2 * _NBUF]
        gsem = bufs[2 * _NBUF:3 * _NBUF]
        wsem = bufs[3 * _NBUF:4 * _NBUF]
        wid = lax.axis_index("s") * _NC + lax.axis_index("c")
        base = wid * per_w

        gcp = [None] * nch
        wcp = [None] * nch
        for ci in range(nch):
            b = ci % _NBUF
            if ci >= _NBUF:
                wcp[ci - _NBUF].wait()
            off = base + ci * _GCH
            pltpu.sync_copy(idx_hbm.at[pl.ds(off, _GCH)], idxs[b])
            gcp[ci] = pltpu.async_copy(g_hbm.at[idxs[b]], rows[b], gsem[b])
            if ci >= _NBUF - 1:
                j = ci - (_NBUF - 1)
                bj = j % _NBUF
                gcp[j].wait()
                wcp[j] = pltpu.async_copy(
                    rows[bj], out_hbm.at[pl.ds(base + j * _GCH, _GCH)],
                    wsem[bj])
        for j in range(nch - (_NBUF - 1), nch):
            b = j % _NBUF
            gcp[j].wait()
            wcp[j] = pltpu.async_copy(
                rows[b], out_hbm.at[pl.ds(base + j * _GCH, _GCH)], wsem[b])
        for j in range(nch - _NBUF, nch):
            wcp[j].wait()

    return gather_k(G, flat_idx)


_QW = N // _NW
_NG = _QW // 16
_UNROLL = 4


def _search(x0, x1, x2, y0, y1, y2):
    mesh = plsc.VectorSubcoreMesh(core_axis_name="c", subcore_axis_name="s")

    @functools.partial(
        pl.kernel, mesh=mesh,
        compiler_params=pltpu.CompilerParams(needs_layout_passes=False),
        out_type=(jax.ShapeDtypeStruct((NE,), jnp.int32),
                  jax.ShapeDtypeStruct((N,), jnp.int32)),
        scratch_types=[
            pltpu.VMEM((N,), jnp.float32),
            pltpu.VMEM((N,), jnp.float32),
            pltpu.VMEM((N,), jnp.float32),
            pltpu.VMEM((_QW,), jnp.float32),
            pltpu.VMEM((_QW,), jnp.float32),
            pltpu.VMEM((_QW,), jnp.float32),
            pltpu.VMEM((_QW * K_CAP,), jnp.int32),
            pltpu.VMEM((_QW,), jnp.int32),
        ],
    )
    def search_k(x0h, x1h, x2h, y0h, y1h, y2h, idxh, cnth,
                 y0v, y1v, y2v, x0v, x1v, x2v, oidx, ocnt):
        wid = lax.axis_index("s") * _NC + lax.axis_index("c")
        qbase = wid * _QW
        pltpu.sync_copy(y0h, y0v)
        pltpu.sync_copy(y1h, y1v)
        pltpu.sync_copy(y2h, y2v)
        pltpu.sync_copy(x0h.at[pl.ds(qbase, _QW)], x0v)
        pltpu.sync_copy(x1h.at[pl.ds(qbase, _QW)], x1v)
        pltpu.sync_copy(x2h.at[pl.ds(qbase, _QW)], x2v)

        lanes = lax.broadcasted_iota(jnp.int32, (16,), 0)
        zeros16 = jnp.zeros((16,), jnp.int32)

        def fill(z, carry):
            oidx[pl.ds(z * 16, 16)] = (lanes + z * 16) & (N - 1)
            return carry
        lax.fori_loop(0, _QW * K_CAP // 16, fill, 0)

        def per_group_pair(g, carry):
            xa = [x0v[pl.ds(g * 32, 16)], x1v[pl.ds(g * 32, 16)],
                  x2v[pl.ds(g * 32, 16)]]
            xb = [x0v[pl.ds(g * 32 + 16, 16)], x1v[pl.ds(g * 32 + 16, 16)],
                  x2v[pl.ds(g * 32 + 16, 16)]]
            rowa = (g * 32 + lanes) * K_CAP
            rowb = (g * 32 + 16 + lanes) * K_CAP

            @plsc.parallel_loop(0, N, step=_UNROLL,
                                carry=(zeros16, zeros16))
            def per_cand(jj, cursors):
                ca, cb = cursors
                for u in range(_UNROLL):
                    jsplat = zeros16 + (jj + u)
                    ys = [plsc.load_gather(y0v, [jsplat]),
                          plsc.load_gather(y1v, [jsplat]),
                          plsc.load_gather(y2v, [jsplat])]
                    da = [ys[c] - xa[c] for c in range(3)]
                    db = [ys[c] - xb[c] for c in range(3)]
                    dista = da[0] * da[0] + da[1] * da[1] + da[2] * da[2]
                    distb = db[0] * db[0] + db[1] * db[1] + db[2] * db[2]
                    oka = jnp.logical_and(dista <= RADIUS2, ca < K_CAP)
                    okb = jnp.logical_and(distb <= RADIUS2, cb < K_CAP)
                    plsc.store_scatter(oidx, [rowa + ca], jsplat, mask=oka)
                    plsc.store_scatter(oidx, [rowb + cb], jsplat, mask=okb)
                    ca = ca + oka.astype(jnp.int32)
                    cb = cb + okb.astype(jnp.int32)
                return ca, cb

            ca, cb = per_cand
            ocnt[pl.ds(g * 32, 16)] = ca
            ocnt[pl.ds(g * 32 + 16, 16)] = cb
            return carry

        lax.fori_loop(0, _NG // 2, per_group_pair, 0)

        pltpu.sync_copy(oidx, idxh.at[pl.ds(qbase * K_CAP, _QW * K_CAP)])
        pltpu.sync_copy(ocnt, cnth.at[pl.ds(qbase, _QW)])

    return search_k(x0, x1, x2, y0, y1, y2)


_BQ = 64


def _mlp_body(e_ref, b_ref, cnt_ref, w1_ref, b1_ref, w2_ref, b2_ref, o_ref):
    e = e_ref[...]
    ea = lax.slice(e, (0, 0), (_BQ * K_CAP, CH))
    ef = lax.slice(e, (0, CH), (_BQ * K_CAP, 2 * CH))
    t = ea.reshape(_BQ, K_CAP, CH) + b_ref[...][:, None, :]
    t = _gelu(t).reshape(_BQ * K_CAP, CH)
    t = _gelu(jnp.dot(t, w1_ref[...], preferred_element_type=jnp.float32)
              + b1_ref[...])
    v = jnp.dot(t, w2_ref[...], preferred_element_type=jnp.float32) + b2_ref[...]
    v = v * ef
    slot = lax.broadcasted_iota(jnp.int32, (_BQ, K_CAP), 1)
    mask = (slot < cnt_ref[...]).astype(jnp.float32)
    v = v.reshape(_BQ, K_CAP, CH) * mask[:, :, None]
    o_ref[...] = jnp.sum(v, axis=1)


def _mlp(E, B, cnt, W1, b1, W2, b2):
    grid = N // _BQ
    return pl.pallas_call(
        _mlp_body,
        grid=(grid,),
        in_specs=[
            pl.BlockSpec((_BQ * K_CAP, 2 * CH), lambda i: (i, 0)),
            pl.BlockSpec((_BQ, CH), lambda i: (i, 0)),
            pl.BlockSpec((_BQ, 1), lambda i: (i, 0)),
            pl.BlockSpec((CH, CH), lambda i: (0, 0)),
            pl.BlockSpec((1, CH), lambda i: (0, 0)),
            pl.BlockSpec((CH, CH), lambda i: (0, 0)),
            pl.BlockSpec((1, CH), lambda i: (0, 0)),
        ],
        out_specs=pl.BlockSpec((_BQ, CH), lambda i: (i, 0)),
        out_shape=jax.ShapeDtypeStruct((N, CH), jnp.float32),
    )(E, B, cnt.reshape(N, 1), W1, b1.reshape(1, CH), W2, b2.reshape(1, CH))


def kernel(y, x, f_y, W0, b0, W1, b1, W2, b2):
    idx_flat, cnt = _search(x[:, 0], x[:, 1], x[:, 2],
                            y[:, 0], y[:, 1], y[:, 2])
    G, B = _prep(y, x, f_y, W0, b0)
    E = _gather_edges(G, idx_flat)
    return _mlp(E, B, cnt, W1, b1, W2, b2)


# speedup vs baseline: 1.6169x; 1.0113x over previous
import functools

import jax
import jax.numpy as jnp
import numpy as np
from jax import lax
from jax.experimental import pallas as pl
from jax.experimental.pallas import tpu as pltpu
from jax.experimental.pallas import tpu_sc as plsc

N = 4096
K_CAP = 48
COORD_DIM = 3
NUM_FREQ = 16
EMB = COORD_DIM * NUM_FREQ * 2
RADIUS2 = 0.1 * 0.1
CH = 64
NE = N * K_CAP

def _gelu(v):
    return v * 0.5 * (1.0 + lax.erf(v * np.float32(1.0 / np.sqrt(2.0))))


def _embed(t):
    fiota = lax.broadcasted_iota(jnp.int32, (1, NUM_FREQ), 1).astype(jnp.float32)
    freqs = jnp.exp(fiota * np.float32(np.log(1.0 / 10000.0) / NUM_FREQ))
    parts = [lax.slice(t, (0, c), (t.shape[0], c + 1)) * freqs
             for c in range(COORD_DIM)]
    prod = jnp.concatenate(parts, axis=1)
    return jnp.concatenate([jnp.sin(prod), jnp.cos(prod)], axis=1)


def _prep_body(y_ref, x_ref, f_ref, w0_ref, b0_ref, g_ref, b_ref):
    w0a = w0_ref[:EMB, :]
    w0b = w0_ref[EMB:, :]
    a = jnp.dot(_embed(y_ref[...]), w0a, preferred_element_type=jnp.float32)
    g_ref[...] = jnp.concatenate([a, f_ref[...]], axis=1)
    b_ref[...] = (jnp.dot(_embed(x_ref[...]), w0b,
                          preferred_element_type=jnp.float32)
                  + b0_ref[...])


def _prep(y, x, f_y, W0, b0):
    return pl.pallas_call(
        _prep_body,
        out_shape=(jax.ShapeDtypeStruct((N, 2 * CH), jnp.float32),
                   jax.ShapeDtypeStruct((N, CH), jnp.float32)),
    )(y, x, f_y, W0, b0.reshape(1, CH))


_NC = 2
_NS = 16
_NW = _NC * _NS
_GCH = 128


_NBUF = 3


def _gather_edges(G, flat_idx):
    per_w = NE // _NW
    nch = per_w // _GCH
    mesh = plsc.VectorSubcoreMesh(core_axis_name="c", subcore_axis_name="s")

    @functools.partial(
        pl.kernel, mesh=mesh,
        out_type=jax.ShapeDtypeStruct((NE, 2 * CH), jnp.float32),
        scratch_types=(
            [pltpu.VMEM((_GCH,), jnp.int32)] * _NBUF
            + [pltpu.VMEM((_GCH, 2 * CH), jnp.float32)] * _NBUF
            + [pltpu.SemaphoreType.DMA] * (2 * _NBUF)
        ),
    )
    def gather_k(g_hbm, idx_hbm, out_hbm, *bufs):
        idxs = bufs[:_NBUF]
        rows = bufs[_NBUF:2 * _NBUF]
        gsem = bufs[2 * _NBUF:3 * _NBUF]
        wsem = bufs[3 * _NBUF:4 * _NBUF]
        wid = lax.axis_index("s") * _NC + lax.axis_index("c")
        base = wid * per_w

        gcp = [None] * nch
        wcp = [None] * nch
        for ci in range(nch):
            b = ci % _NBUF
            if ci >= _NBUF:
                wcp[ci - _NBUF].wait()
            off = base + ci * _GCH
            pltpu.sync_copy(idx_hbm.at[pl.ds(off, _GCH)], idxs[b])
            gcp[ci] = pltpu.async_copy(g_hbm.at[idxs[b]], rows[b], gsem[b])
            if ci >= _NBUF - 1:
                j = ci - (_NBUF - 1)
                bj = j % _NBUF
                gcp[j].wait()
                wcp[j] = pltpu.async_copy(
                    rows[bj], out_hbm.at[pl.ds(base + j * _GCH, _GCH)],
                    wsem[bj])
        for j in range(nch - (_NBUF - 1), nch):
            b = j % _NBUF
            gcp[j].wait()
            wcp[j] = pltpu.async_copy(
                rows[b], out_hbm.at[pl.ds(base + j * _GCH, _GCH)], wsem[b])
        for j in range(nch - _NBUF, nch):
            wcp[j].wait()

    return gather_k(G, flat_idx)


_QW = N // _NW
_NG = _QW // 16
_UNROLL = 4
_ROW = 64


def _search(x0, x1, x2, y0, y1, y2):
    mesh = plsc.VectorSubcoreMesh(core_axis_name="c", subcore_axis_name="s")

    @functools.partial(
        pl.kernel, mesh=mesh,
        compiler_params=pltpu.CompilerParams(needs_layout_passes=False),
        out_type=(jax.ShapeDtypeStruct((NE,), jnp.int32),
                  jax.ShapeDtypeStruct((N,), jnp.int32)),
        scratch_types=[
            pltpu.VMEM((N,), jnp.float32),
            pltpu.VMEM((N,), jnp.float32),
            pltpu.VMEM((N,), jnp.float32),
            pltpu.VMEM((_QW,), jnp.float32),
            pltpu.VMEM((_QW,), jnp.float32),
            pltpu.VMEM((_QW,), jnp.float32),
            pltpu.VMEM((_QW * _ROW,), jnp.int32),
            pltpu.VMEM((_QW * K_CAP,), jnp.int32),
            pltpu.VMEM((_QW,), jnp.int32),
        ],
    )
    def search_k(x0h, x1h, x2h, y0h, y1h, y2h, idxh, cnth,
                 y0v, y1v, y2v, x0v, x1v, x2v, oidx, cidx, ocnt):
        wid = lax.axis_index("s") * _NC + lax.axis_index("c")
        qbase = wid * _QW
        pltpu.sync_copy(y0h, y0v)
        pltpu.sync_copy(y1h, y1v)
        pltpu.sync_copy(y2h, y2v)
        pltpu.sync_copy(x0h.at[pl.ds(qbase, _QW)], x0v)
        pltpu.sync_copy(x1h.at[pl.ds(qbase, _QW)], x1v)
        pltpu.sync_copy(x2h.at[pl.ds(qbase, _QW)], x2v)

        lanes = lax.broadcasted_iota(jnp.int32, (16,), 0)
        zeros16 = jnp.zeros((16,), jnp.int32)

        def fill(z, carry):
            oidx[pl.ds(z * 16, 16)] = (lanes + z * 16) & (N - 1)
            return carry
        lax.fori_loop(0, _QW * _ROW // 16, fill, 0)

        def per_group_pair(g, carry):
            xa = [x0v[pl.ds(g * 32, 16)], x1v[pl.ds(g * 32, 16)],
                  x2v[pl.ds(g * 32, 16)]]
            xb = [x0v[pl.ds(g * 32 + 16, 16)], x1v[pl.ds(g * 32 + 16, 16)],
                  x2v[pl.ds(g * 32 + 16, 16)]]
            rowa = (g * 32 + lanes) * _ROW
            rowb = (g * 32 + 16 + lanes) * _ROW

            @plsc.parallel_loop(0, N, step=_UNROLL,
                                carry=(zeros16, zeros16))
            def per_cand(jj, cursors):
                ca, cb = cursors
                for u in range(_UNROLL):
                    jsplat = zeros16 + (jj + u)
                    ys = [plsc.load_gather(y0v, [jsplat]),
                          plsc.load_gather(y1v, [jsplat]),
                          plsc.load_gather(y2v, [jsplat])]
                    da = [ys[c] - xa[c] for c in range(3)]
                    db = [ys[c] - xb[c] for c in range(3)]
                    dista = da[0] * da[0] + da[1] * da[1] + da[2] * da[2]
                    distb = db[0] * db[0] + db[1] * db[1] + db[2] * db[2]
                    oka = dista <= RADIUS2
                    okb = distb <= RADIUS2
                    plsc.store_scatter(
                        oidx, [rowa + jnp.minimum(ca, _ROW - 1)], jsplat,
                        mask=oka)
                    plsc.store_scatter(
                        oidx, [rowb + jnp.minimum(cb, _ROW - 1)], jsplat,
                        mask=okb)
                    ca = ca + oka.astype(jnp.int32)
                    cb = cb + okb.astype(jnp.int32)
                return ca, cb

            ca, cb = per_cand
            ocnt[pl.ds(g * 32, 16)] = jnp.minimum(ca, K_CAP)
            ocnt[pl.ds(g * 32 + 16, 16)] = jnp.minimum(cb, K_CAP)
            return carry

        lax.fori_loop(0, _NG // 2, per_group_pair, 0)

        def compact(q, carry):
            for k in range(K_CAP // 16):
                cidx[pl.ds(q * K_CAP + k * 16, 16)] = (
                    oidx[pl.ds(q * _ROW + k * 16, 16)])
            return carry
        lax.fori_loop(0, _QW, compact, 0)

        pltpu.sync_copy(cidx, idxh.at[pl.ds(qbase * K_CAP, _QW * K_CAP)])
        pltpu.sync_copy(ocnt, cnth.at[pl.ds(qbase, _QW)])

    return search_k(x0, x1, x2, y0, y1, y2)


_BQ = 64


def _mlp_body(e_ref, b_ref, cnt_ref, w1_ref, b1_ref, w2_ref, b2_ref, o_ref):
    e = e_ref[...]
    ea = lax.slice(e, (0, 0), (_BQ * K_CAP, CH))
    ef = lax.slice(e, (0, CH), (_BQ * K_CAP, 2 * CH))
    t = ea.reshape(_BQ, K_CAP, CH) + b_ref[...][:, None, :]
    t = _gelu(t).reshape(_BQ * K_CAP, CH)
    t = _gelu(jnp.dot(t, w1_ref[...], preferred_element_type=jnp.float32)
              + b1_ref[...])
    v = jnp.dot(t, w2_ref[...], preferred_element_type=jnp.float32) + b2_ref[...]
    v = v * ef
    slot = lax.broadcasted_iota(jnp.int32, (_BQ, K_CAP), 1)
    mask = (slot < cnt_ref[...]).astype(jnp.float32)
    v = v.reshape(_BQ, K_CAP, CH) * mask[:, :, None]
    o_ref[...] = jnp.sum(v, axis=1)


def _mlp(E, B, cnt, W1, b1, W2, b2):
    grid = N // _BQ
    return pl.pallas_call(
        _mlp_body,
        grid=(grid,),
        in_specs=[
            pl.BlockSpec((_BQ * K_CAP, 2 * CH), lambda i: (i, 0)),
            pl.BlockSpec((_BQ, CH), lambda i: (i, 0)),
            pl.BlockSpec((_BQ, 1), lambda i: (i, 0)),
            pl.BlockSpec((CH, CH), lambda i: (0, 0)),
            pl.BlockSpec((1, CH), lambda i: (0, 0)),
            pl.BlockSpec((CH, CH), lambda i: (0, 0)),
            pl.BlockSpec((1, CH), lambda i: (0, 0)),
        ],
        out_specs=pl.BlockSpec((_BQ, CH), lambda i: (i, 0)),
        out_shape=jax.ShapeDtypeStruct((N, CH), jnp.float32),
    )(E, B, cnt.reshape(N, 1), W1, b1.reshape(1, CH), W2, b2.reshape(1, CH))


def kernel(y, x, f_y, W0, b0, W1, b1, W2, b2):
    idx_flat, cnt = _search(x[:, 0], x[:, 1], x[:, 2],
                            y[:, 0], y[:, 1], y[:, 2])
    G, B = _prep(y, x, f_y, W0, b0)
    E = _gather_edges(G, idx_flat)
    return _mlp(E, B, cnt, W1, b1, W2, b2)


# speedup vs baseline: 1.8102x; 1.1196x over previous
import functools

import jax
import jax.numpy as jnp
import numpy as np
from jax import lax
from jax.experimental import pallas as pl
from jax.experimental.pallas import tpu as pltpu
from jax.experimental.pallas import tpu_sc as plsc

N = 4096
K_CAP = 48
COORD_DIM = 3
NUM_FREQ = 16
EMB = COORD_DIM * NUM_FREQ * 2
RADIUS2 = 0.1 * 0.1
CH = 64
NE = N * K_CAP

def _gelu(v):
    return v * 0.5 * (1.0 + lax.erf(v * np.float32(1.0 / np.sqrt(2.0))))


def _embed(t):
    fiota = lax.broadcasted_iota(jnp.int32, (1, NUM_FREQ), 1).astype(jnp.float32)
    freqs = jnp.exp(fiota * np.float32(np.log(1.0 / 10000.0) / NUM_FREQ))
    parts = [lax.slice(t, (0, c), (t.shape[0], c + 1)) * freqs
             for c in range(COORD_DIM)]
    prod = jnp.concatenate(parts, axis=1)
    return jnp.concatenate([jnp.sin(prod), jnp.cos(prod)], axis=1)


def _prep_body(y_ref, x_ref, f_ref, w0_ref, b0_ref, g_ref, b_ref):
    w0a = w0_ref[:EMB, :]
    w0b = w0_ref[EMB:, :]
    a = jnp.dot(_embed(y_ref[...]), w0a, preferred_element_type=jnp.float32)
    g_ref[...] = jnp.concatenate([a, f_ref[...]], axis=1)
    b_ref[...] = (jnp.dot(_embed(x_ref[...]), w0b,
                          preferred_element_type=jnp.float32)
                  + b0_ref[...])


def _prep(y, x, f_y, W0, b0):
    return pl.pallas_call(
        _prep_body,
        out_shape=(jax.ShapeDtypeStruct((N, 2 * CH), jnp.float32),
                   jax.ShapeDtypeStruct((N, CH), jnp.float32)),
    )(y, x, f_y, W0, b0.reshape(1, CH))


_NC = 2
_NS = 16
_NW = _NC * _NS
_GCH = 128


_NBUF = 3


def _gather_edges(G, flat_idx):
    per_w = NE // _NW
    nch = per_w // _GCH
    mesh = plsc.VectorSubcoreMesh(core_axis_name="c", subcore_axis_name="s")

    @functools.partial(
        pl.kernel, mesh=mesh,
        out_type=jax.ShapeDtypeStruct((NE, 2 * CH), jnp.float32),
        scratch_types=(
            [pltpu.VMEM((_GCH,), jnp.int32)] * _NBUF
            + [pltpu.VMEM((_GCH, 2 * CH), jnp.float32)] * _NBUF
            + [pltpu.SemaphoreType.DMA] * (2 * _NBUF)
        ),
    )
    def gather_k(g_hbm, idx_hbm, out_hbm, *bufs):
        idxs = bufs[:_NBUF]
        rows = bufs[_NBUF:2 * _NBUF]
        gsem = bufs[2 * _NBUF:3 * _NBUF]
        wsem = bufs[3 * _NBUF:4 * _NBUF]
        wid = lax.axis_index("s") * _NC + lax.axis_index("c")
        base = wid * per_w

        gcp = [None] * nch
        wcp = [None] * nch
        for ci in range(nch):
            b = ci % _NBUF
            if ci >= _NBUF:
                wcp[ci - _NBUF].wait()
            off = base + ci * _GCH
            pltpu.sync_copy(idx_hbm.at[pl.ds(off, _GCH)], idxs[b])
            gcp[ci] = pltpu.async_copy(g_hbm.at[idxs[b]], rows[b], gsem[b])
            if ci >= _NBUF - 1:
                j = ci - (_NBUF - 1)
                bj = j % _NBUF
                gcp[j].wait()
                wcp[j] = pltpu.async_copy(
                    rows[bj], out_hbm.at[pl.ds(base + j * _GCH, _GCH)],
                    wsem[bj])
        for j in range(nch - (_NBUF - 1), nch):
            b = j % _NBUF
            gcp[j].wait()
            wcp[j] = pltpu.async_copy(
                rows[b], out_hbm.at[pl.ds(base + j * _GCH, _GCH)], wsem[b])
        for j in range(nch - _NBUF, nch):
            wcp[j].wait()

    return gather_k(G, flat_idx)


_QW = N // _NW
_NG = _QW // 16
_UNROLL = 2
_ROW = 64


def _search(x0, x1, x2, y0, y1, y2):
    mesh = plsc.VectorSubcoreMesh(core_axis_name="c", subcore_axis_name="s")

    @functools.partial(
        pl.kernel, mesh=mesh,
        compiler_params=pltpu.CompilerParams(needs_layout_passes=False),
        out_type=(jax.ShapeDtypeStruct((NE,), jnp.int32),
                  jax.ShapeDtypeStruct((N,), jnp.int32)),
        scratch_types=[
            pltpu.VMEM((N,), jnp.float32),
            pltpu.VMEM((N,), jnp.float32),
            pltpu.VMEM((N,), jnp.float32),
            pltpu.VMEM((_QW,), jnp.float32),
            pltpu.VMEM((_QW,), jnp.float32),
            pltpu.VMEM((_QW,), jnp.float32),
            pltpu.VMEM((_QW * _ROW,), jnp.int32),
            pltpu.VMEM((_QW * K_CAP,), jnp.int32),
            pltpu.VMEM((_QW,), jnp.int32),
        ],
    )
    def search_k(x0h, x1h, x2h, y0h, y1h, y2h, idxh, cnth,
                 y0v, y1v, y2v, x0v, x1v, x2v, oidx, cidx, ocnt):
        wid = lax.axis_index("s") * _NC + lax.axis_index("c")
        qbase = wid * _QW
        pltpu.sync_copy(y0h, y0v)
        pltpu.sync_copy(y1h, y1v)
        pltpu.sync_copy(y2h, y2v)
        pltpu.sync_copy(x0h.at[pl.ds(qbase, _QW)], x0v)
        pltpu.sync_copy(x1h.at[pl.ds(qbase, _QW)], x1v)
        pltpu.sync_copy(x2h.at[pl.ds(qbase, _QW)], x2v)

        lanes = lax.broadcasted_iota(jnp.int32, (16,), 0)
        zeros16 = jnp.zeros((16,), jnp.int32)

        def fill(z, carry):
            oidx[pl.ds(z * 16, 16)] = (lanes + z * 16) & (N - 1)
            return carry
        lax.fori_loop(0, _QW * _ROW // 16, fill, 0)

        def per_group_pair(g, carry):
            xa = [x0v[pl.ds(g * 32, 16)], x1v[pl.ds(g * 32, 16)],
                  x2v[pl.ds(g * 32, 16)]]
            xb = [x0v[pl.ds(g * 32 + 16, 16)], x1v[pl.ds(g * 32 + 16, 16)],
                  x2v[pl.ds(g * 32 + 16, 16)]]
            rowa = (g * 32 + lanes) * _ROW
            rowb = (g * 32 + 16 + lanes) * _ROW

            @plsc.parallel_loop(0, N, step=_UNROLL,
                                carry=(zeros16, zeros16))
            def per_cand(jj, cursors):
                ca, cb = cursors
                for u in range(_UNROLL):
                    jsplat = zeros16 + (jj + u)
                    ys = [plsc.load_gather(y0v, [jsplat]),
                          plsc.load_gather(y1v, [jsplat]),
                          plsc.load_gather(y2v, [jsplat])]
                    da = [ys[c] - xa[c] for c in range(3)]
                    db = [ys[c] - xb[c] for c in range(3)]
                    dista = da[0] * da[0] + da[1] * da[1] + da[2] * da[2]
                    distb = db[0] * db[0] + db[1] * db[1] + db[2] * db[2]
                    oka = dista <= RADIUS2
                    okb = distb <= RADIUS2
                    plsc.store_scatter(
                        oidx, [rowa + jnp.minimum(ca, _ROW - 1)], jsplat,
                        mask=oka)
                    plsc.store_scatter(
                        oidx, [rowb + jnp.minimum(cb, _ROW - 1)], jsplat,
                        mask=okb)
                    ca = ca + oka.astype(jnp.int32)
                    cb = cb + okb.astype(jnp.int32)
                return ca, cb

            ca, cb = per_cand
            ocnt[pl.ds(g * 32, 16)] = jnp.minimum(ca, K_CAP)
            ocnt[pl.ds(g * 32 + 16, 16)] = jnp.minimum(cb, K_CAP)
            return carry

        lax.fori_loop(0, _NG // 2, per_group_pair, 0)

        def compact(q, carry):
            for k in range(K_CAP // 16):
                cidx[pl.ds(q * K_CAP + k * 16, 16)] = (
                    oidx[pl.ds(q * _ROW + k * 16, 16)])
            return carry
        lax.fori_loop(0, _QW, compact, 0)

        pltpu.sync_copy(cidx, idxh.at[pl.ds(qbase * K_CAP, _QW * K_CAP)])
        pltpu.sync_copy(ocnt, cnth.at[pl.ds(qbase, _QW)])

    return search_k(x0, x1, x2, y0, y1, y2)


_BQ = 64


def _mlp_body(e_ref, b_ref, cnt_ref, w1_ref, b1_ref, w2_ref, b2_ref, o_ref):
    e = e_ref[...]
    ea = lax.slice(e, (0, 0), (_BQ * K_CAP, CH))
    ef = lax.slice(e, (0, CH), (_BQ * K_CAP, 2 * CH))
    t = ea.reshape(_BQ, K_CAP, CH) + b_ref[...][:, None, :]
    t = _gelu(t).reshape(_BQ * K_CAP, CH)
    t = _gelu(jnp.dot(t, w1_ref[...], preferred_element_type=jnp.float32)
              + b1_ref[...])
    v = jnp.dot(t, w2_ref[...], preferred_element_type=jnp.float32) + b2_ref[...]
    v = v * ef
    slot = lax.broadcasted_iota(jnp.int32, (_BQ, K_CAP), 1)
    mask = (slot < cnt_ref[...]).astype(jnp.float32)
    v = v.reshape(_BQ, K_CAP, CH) * mask[:, :, None]
    o_ref[...] = jnp.sum(v, axis=1)


def _mlp(E, B, cnt, W1, b1, W2, b2):
    grid = N // _BQ
    return pl.pallas_call(
        _mlp_body,
        grid=(grid,),
        in_specs=[
            pl.BlockSpec((_BQ * K_CAP, 2 * CH), lambda i: (i, 0)),
            pl.BlockSpec((_BQ, CH), lambda i: (i, 0)),
            pl.BlockSpec((_BQ, 1), lambda i: (i, 0)),
            pl.BlockSpec((CH, CH), lambda i: (0, 0)),
            pl.BlockSpec((1, CH), lambda i: (0, 0)),
            pl.BlockSpec((CH, CH), lambda i: (0, 0)),
            pl.BlockSpec((1, CH), lambda i: (0, 0)),
        ],
        out_specs=pl.BlockSpec((_BQ, CH), lambda i: (i, 0)),
        out_shape=jax.ShapeDtypeStruct((N, CH), jnp.float32),
    )(E, B, cnt.reshape(N, 1), W1, b1.reshape(1, CH), W2, b2.reshape(1, CH))


def kernel(y, x, f_y, W0, b0, W1, b1, W2, b2):
    idx_flat, cnt = _search(x[:, 0], x[:, 1], x[:, 2],
                            y[:, 0], y[:, 1], y[:, 2])
    G, B = _prep(y, x, f_y, W0, b0)
    E = _gather_edges(G, idx_flat)
    return _mlp(E, B, cnt, W1, b1, W2, b2)


# speedup vs baseline: 1.9870x; 1.0976x over previous
import functools

import jax
import jax.numpy as jnp
import numpy as np
from jax import lax
from jax.experimental import pallas as pl
from jax.experimental.pallas import tpu as pltpu
from jax.experimental.pallas import tpu_sc as plsc

N = 4096
K_CAP = 48
COORD_DIM = 3
NUM_FREQ = 16
EMB = COORD_DIM * NUM_FREQ * 2
RADIUS2 = 0.1 * 0.1
CH = 64
NE = N * K_CAP

def _gelu(v):
    return v * 0.5 * (1.0 + lax.erf(v * np.float32(1.0 / np.sqrt(2.0))))


def _embed(t):
    fiota = lax.broadcasted_iota(jnp.int32, (1, NUM_FREQ), 1).astype(jnp.float32)
    freqs = jnp.exp(fiota * np.float32(np.log(1.0 / 10000.0) / NUM_FREQ))
    parts = [lax.slice(t, (0, c), (t.shape[0], c + 1)) * freqs
             for c in range(COORD_DIM)]
    prod = jnp.concatenate(parts, axis=1)
    return jnp.concatenate([jnp.sin(prod), jnp.cos(prod)], axis=1)


def _prep_body(y_ref, x_ref, f_ref, w0_ref, b0_ref, g_ref, b_ref):
    w0a = w0_ref[:EMB, :]
    w0b = w0_ref[EMB:, :]
    a = jnp.dot(_embed(y_ref[...]), w0a, preferred_element_type=jnp.float32)
    g_ref[...] = jnp.concatenate([a, f_ref[...]], axis=1)
    b_ref[...] = (jnp.dot(_embed(x_ref[...]), w0b,
                          preferred_element_type=jnp.float32)
                  + b0_ref[...])


def _prep(y, x, f_y, W0, b0):
    return pl.pallas_call(
        _prep_body,
        out_shape=(jax.ShapeDtypeStruct((N, 2 * CH), jnp.float32),
                   jax.ShapeDtypeStruct((N, CH), jnp.float32)),
    )(y, x, f_y, W0, b0.reshape(1, CH))


_NC = 2
_NS = 16
_NW = _NC * _NS
_GCH = 128


_NBUF = 3


def _gather_edges(G, flat_idx):
    per_w = NE // _NW
    nch = per_w // _GCH
    mesh = plsc.VectorSubcoreMesh(core_axis_name="c", subcore_axis_name="s")

    @functools.partial(
        pl.kernel, mesh=mesh,
        out_type=jax.ShapeDtypeStruct((NE, 2 * CH), jnp.float32),
        scratch_types=(
            [pltpu.VMEM((_GCH,), jnp.int32)] * _NBUF
            + [pltpu.VMEM((_GCH, 2 * CH), jnp.float32)] * _NBUF
            + [pltpu.SemaphoreType.DMA] * (2 * _NBUF)
        ),
    )
    def gather_k(g_hbm, idx_hbm, out_hbm, *bufs):
        idxs = bufs[:_NBUF]
        rows = bufs[_NBUF:2 * _NBUF]
        gsem = bufs[2 * _NBUF:3 * _NBUF]
        wsem = bufs[3 * _NBUF:4 * _NBUF]
        wid = lax.axis_index("s") * _NC + lax.axis_index("c")
        base = wid * per_w

        gcp = [None] * nch
        wcp = [None] * nch
        for ci in range(nch):
            b = ci % _NBUF
            if ci >= _NBUF:
                wcp[ci - _NBUF].wait()
            off = base + ci * _GCH
            pltpu.sync_copy(idx_hbm.at[pl.ds(off, _GCH)], idxs[b])
            gcp[ci] = pltpu.async_copy(g_hbm.at[idxs[b]], rows[b], gsem[b])
            if ci >= _NBUF - 1:
                j = ci - (_NBUF - 1)
                bj = j % _NBUF
                gcp[j].wait()
                wcp[j] = pltpu.async_copy(
                    rows[bj], out_hbm.at[pl.ds(base + j * _GCH, _GCH)],
                    wsem[bj])
        for j in range(nch - (_NBUF - 1), nch):
            b = j % _NBUF
            gcp[j].wait()
            wcp[j] = pltpu.async_copy(
                rows[b], out_hbm.at[pl.ds(base + j * _GCH, _GCH)], wsem[b])
        for j in range(nch - _NBUF, nch):
            wcp[j].wait()

    return gather_k(G, flat_idx)


_QW = N // _NW
_NG = _QW // 16
_UNROLL = 1
_ROW = 64


def _search(x0, x1, x2, y0, y1, y2):
    mesh = plsc.VectorSubcoreMesh(core_axis_name="c", subcore_axis_name="s")

    @functools.partial(
        pl.kernel, mesh=mesh,
        compiler_params=pltpu.CompilerParams(needs_layout_passes=False),
        out_type=(jax.ShapeDtypeStruct((NE,), jnp.int32),
                  jax.ShapeDtypeStruct((N,), jnp.int32)),
        scratch_types=[
            pltpu.VMEM((N,), jnp.float32),
            pltpu.VMEM((N,), jnp.float32),
            pltpu.VMEM((N,), jnp.float32),
            pltpu.VMEM((_QW,), jnp.float32),
            pltpu.VMEM((_QW,), jnp.float32),
            pltpu.VMEM((_QW,), jnp.float32),
            pltpu.VMEM((_QW * _ROW,), jnp.int32),
            pltpu.VMEM((_QW * K_CAP,), jnp.int32),
            pltpu.VMEM((_QW,), jnp.int32),
        ],
    )
    def search_k(x0h, x1h, x2h, y0h, y1h, y2h, idxh, cnth,
                 y0v, y1v, y2v, x0v, x1v, x2v, oidx, cidx, ocnt):
        wid = lax.axis_index("s") * _NC + lax.axis_index("c")
        qbase = wid * _QW
        pltpu.sync_copy(y0h, y0v)
        pltpu.sync_copy(y1h, y1v)
        pltpu.sync_copy(y2h, y2v)
        pltpu.sync_copy(x0h.at[pl.ds(qbase, _QW)], x0v)
        pltpu.sync_copy(x1h.at[pl.ds(qbase, _QW)], x1v)
        pltpu.sync_copy(x2h.at[pl.ds(qbase, _QW)], x2v)

        lanes = lax.broadcasted_iota(jnp.int32, (16,), 0)
        zeros16 = jnp.zeros((16,), jnp.int32)

        def fill(z, carry):
            oidx[pl.ds(z * 16, 16)] = (lanes + z * 16) & (N - 1)
            return carry
        lax.fori_loop(0, _QW * _ROW // 16, fill, 0)

        def per_group_pair(g, carry):
            xa = [x0v[pl.ds(g * 32, 16)], x1v[pl.ds(g * 32, 16)],
                  x2v[pl.ds(g * 32, 16)]]
            xb = [x0v[pl.ds(g * 32 + 16, 16)], x1v[pl.ds(g * 32 + 16, 16)],
                  x2v[pl.ds(g * 32 + 16, 16)]]
            rowa = (g * 32 + lanes) * _ROW
            rowb = (g * 32 + 16 + lanes) * _ROW

            @plsc.parallel_loop(0, N, step=_UNROLL,
                                carry=(zeros16, zeros16))
            def per_cand(jj, cursors):
                ca, cb = cursors
                for u in range(_UNROLL):
                    jsplat = zeros16 + (jj + u)
                    ys = [plsc.load_gather(y0v, [jsplat]),
                          plsc.load_gather(y1v, [jsplat]),
                          plsc.load_gather(y2v, [jsplat])]
                    da = [ys[c] - xa[c] for c in range(3)]
                    db = [ys[c] - xb[c] for c in range(3)]
                    dista = da[0] * da[0] + da[1] * da[1] + da[2] * da[2]
                    distb = db[0] * db[0] + db[1] * db[1] + db[2] * db[2]
                    oka = dista <= RADIUS2
                    okb = distb <= RADIUS2
                    plsc.store_scatter(
                        oidx, [rowa + jnp.minimum(ca, _ROW - 1)], jsplat,
                        mask=oka)
                    plsc.store_scatter(
                        oidx, [rowb + jnp.minimum(cb, _ROW - 1)], jsplat,
                        mask=okb)
                    ca = ca + oka.astype(jnp.int32)
                    cb = cb + okb.astype(jnp.int32)
                return ca, cb

            ca, cb = per_cand
            ocnt[pl.ds(g * 32, 16)] = jnp.minimum(ca, K_CAP)
            ocnt[pl.ds(g * 32 + 16, 16)] = jnp.minimum(cb, K_CAP)
            return carry

        lax.fori_loop(0, _NG // 2, per_group_pair, 0)

        def compact(q, carry):
            for k in range(K_CAP // 16):
                cidx[pl.ds(q * K_CAP + k * 16, 16)] = (
                    oidx[pl.ds(q * _ROW + k * 16, 16)])
            return carry
        lax.fori_loop(0, _QW, compact, 0)

        pltpu.sync_copy(cidx, idxh.at[pl.ds(qbase * K_CAP, _QW * K_CAP)])
        pltpu.sync_copy(ocnt, cnth.at[pl.ds(qbase, _QW)])

    return search_k(x0, x1, x2, y0, y1, y2)


_BQ = 64


def _mlp_body(e_ref, b_ref, cnt_ref, w1_ref, b1_ref, w2_ref, b2_ref, o_ref):
    e = e_ref[...]
    ea = lax.slice(e, (0, 0), (_BQ * K_CAP, CH))
    ef = lax.slice(e, (0, CH), (_BQ * K_CAP, 2 * CH))
    t = ea.reshape(_BQ, K_CAP, CH) + b_ref[...][:, None, :]
    t = _gelu(t).reshape(_BQ * K_CAP, CH)
    t = _gelu(jnp.dot(t, w1_ref[...], preferred_element_type=jnp.float32)
              + b1_ref[...])
    v = jnp.dot(t, w2_ref[...], preferred_element_type=jnp.float32) + b2_ref[...]
    v = v * ef
    slot = lax.broadcasted_iota(jnp.int32, (_BQ, K_CAP), 1)
    mask = (slot < cnt_ref[...]).astype(jnp.float32)
    v = v.reshape(_BQ, K_CAP, CH) * mask[:, :, None]
    o_ref[...] = jnp.sum(v, axis=1)


def _mlp(E, B, cnt, W1, b1, W2, b2):
    grid = N // _BQ
    return pl.pallas_call(
        _mlp_body,
        grid=(grid,),
        in_specs=[
            pl.BlockSpec((_BQ * K_CAP, 2 * CH), lambda i: (i, 0)),
            pl.BlockSpec((_BQ, CH), lambda i: (i, 0)),
            pl.BlockSpec((_BQ, 1), lambda i: (i, 0)),
            pl.BlockSpec((CH, CH), lambda i: (0, 0)),
            pl.BlockSpec((1, CH), lambda i: (0, 0)),
            pl.BlockSpec((CH, CH), lambda i: (0, 0)),
            pl.BlockSpec((1, CH), lambda i: (0, 0)),
        ],
        out_specs=pl.BlockSpec((_BQ, CH), lambda i: (i, 0)),
        out_shape=jax.ShapeDtypeStruct((N, CH), jnp.float32),
    )(E, B, cnt.reshape(N, 1), W1, b1.reshape(1, CH), W2, b2.reshape(1, CH))


def kernel(y, x, f_y, W0, b0, W1, b1, W2, b2):
    idx_flat, cnt = _search(x[:, 0], x[:, 1], x[:, 2],
                            y[:, 0], y[:, 1], y[:, 2])
    G, B = _prep(y, x, f_y, W0, b0)
    E = _gather_edges(G, idx_flat)
    return _mlp(E, B, cnt, W1, b1, W2, b2)


# speedup vs baseline: 2.0220x; 1.0176x over previous
import functools

import jax
import jax.numpy as jnp
import numpy as np
from jax import lax
from jax.experimental import pallas as pl
from jax.experimental.pallas import tpu as pltpu
from jax.experimental.pallas import tpu_sc as plsc

N = 4096
K_CAP = 48
COORD_DIM = 3
NUM_FREQ = 16
EMB = COORD_DIM * NUM_FREQ * 2
RADIUS2 = 0.1 * 0.1
CH = 64
NE = N * K_CAP

def _gelu(v):
    return v * 0.5 * (1.0 + lax.erf(v * np.float32(1.0 / np.sqrt(2.0))))


def _embed(t):
    fiota = lax.broadcasted_iota(jnp.int32, (1, NUM_FREQ), 1).astype(jnp.float32)
    freqs = jnp.exp(fiota * np.float32(np.log(1.0 / 10000.0) / NUM_FREQ))
    parts = [lax.slice(t, (0, c), (t.shape[0], c + 1)) * freqs
             for c in range(COORD_DIM)]
    prod = jnp.concatenate(parts, axis=1)
    return jnp.concatenate([jnp.sin(prod), jnp.cos(prod)], axis=1)


def _prep_body(y_ref, x_ref, f_ref, w0_ref, b0_ref, g_ref, b_ref):
    w0a = w0_ref[:EMB, :]
    w0b = w0_ref[EMB:, :]
    a = jnp.dot(_embed(y_ref[...]), w0a, preferred_element_type=jnp.float32)
    g_ref[...] = jnp.concatenate([a, f_ref[...]], axis=1)
    b_ref[...] = (jnp.dot(_embed(x_ref[...]), w0b,
                          preferred_element_type=jnp.float32)
                  + b0_ref[...])


def _prep(y, x, f_y, W0, b0):
    return pl.pallas_call(
        _prep_body,
        out_shape=(jax.ShapeDtypeStruct((N, 2 * CH), jnp.float32),
                   jax.ShapeDtypeStruct((N, CH), jnp.float32)),
    )(y, x, f_y, W0, b0.reshape(1, CH))


_NC = 2
_NS = 16
_NW = _NC * _NS
_GCH = 128


_NBUF = 3


def _gather_edges(G, flat_idx):
    per_w = NE // _NW
    nch = per_w // _GCH
    mesh = plsc.VectorSubcoreMesh(core_axis_name="c", subcore_axis_name="s")

    @functools.partial(
        pl.kernel, mesh=mesh,
        out_type=jax.ShapeDtypeStruct((NE, 2 * CH), jnp.float32),
        scratch_types=(
            [pltpu.VMEM((_GCH,), jnp.int32)] * _NBUF
            + [pltpu.VMEM((_GCH, 2 * CH), jnp.float32)] * _NBUF
            + [pltpu.SemaphoreType.DMA] * (2 * _NBUF)
        ),
    )
    def gather_k(g_hbm, idx_hbm, out_hbm, *bufs):
        idxs = bufs[:_NBUF]
        rows = bufs[_NBUF:2 * _NBUF]
        gsem = bufs[2 * _NBUF:3 * _NBUF]
        wsem = bufs[3 * _NBUF:4 * _NBUF]
        wid = lax.axis_index("s") * _NC + lax.axis_index("c")
        base = wid * per_w

        gcp = [None] * nch
        wcp = [None] * nch
        for ci in range(nch):
            b = ci % _NBUF
            if ci >= _NBUF:
                wcp[ci - _NBUF].wait()
            off = base + ci * _GCH
            pltpu.sync_copy(idx_hbm.at[pl.ds(off, _GCH)], idxs[b])
            gcp[ci] = pltpu.async_copy(g_hbm.at[idxs[b]], rows[b], gsem[b])
            if ci >= _NBUF - 1:
                j = ci - (_NBUF - 1)
                bj = j % _NBUF
                gcp[j].wait()
                wcp[j] = pltpu.async_copy(
                    rows[bj], out_hbm.at[pl.ds(base + j * _GCH, _GCH)],
                    wsem[bj])
        for j in range(nch - (_NBUF - 1), nch):
            b = j % _NBUF
            gcp[j].wait()
            wcp[j] = pltpu.async_copy(
                rows[b], out_hbm.at[pl.ds(base + j * _GCH, _GCH)], wsem[b])
        for j in range(nch - _NBUF, nch):
            wcp[j].wait()

    return gather_k(G, flat_idx)


_QW = N // _NW
_NG = _QW // 16
_UNROLL = 1
_GRP = 4
_ROW = 64


def _search(x0, x1, x2, y0, y1, y2):
    mesh = plsc.VectorSubcoreMesh(core_axis_name="c", subcore_axis_name="s")

    @functools.partial(
        pl.kernel, mesh=mesh,
        compiler_params=pltpu.CompilerParams(needs_layout_passes=False),
        out_type=(jax.ShapeDtypeStruct((NE,), jnp.int32),
                  jax.ShapeDtypeStruct((N,), jnp.int32)),
        scratch_types=[
            pltpu.VMEM((N,), jnp.float32),
            pltpu.VMEM((N,), jnp.float32),
            pltpu.VMEM((N,), jnp.float32),
            pltpu.VMEM((_QW,), jnp.float32),
            pltpu.VMEM((_QW,), jnp.float32),
            pltpu.VMEM((_QW,), jnp.float32),
            pltpu.VMEM((_QW * _ROW,), jnp.int32),
            pltpu.VMEM((_QW * K_CAP,), jnp.int32),
            pltpu.VMEM((_QW,), jnp.int32),
        ],
    )
    def search_k(x0h, x1h, x2h, y0h, y1h, y2h, idxh, cnth,
                 y0v, y1v, y2v, x0v, x1v, x2v, oidx, cidx, ocnt):
        wid = lax.axis_index("s") * _NC + lax.axis_index("c")
        qbase = wid * _QW
        pltpu.sync_copy(y0h, y0v)
        pltpu.sync_copy(y1h, y1v)
        pltpu.sync_copy(y2h, y2v)
        pltpu.sync_copy(x0h.at[pl.ds(qbase, _QW)], x0v)
        pltpu.sync_copy(x1h.at[pl.ds(qbase, _QW)], x1v)
        pltpu.sync_copy(x2h.at[pl.ds(qbase, _QW)], x2v)

        lanes = lax.broadcasted_iota(jnp.int32, (16,), 0)
        zeros16 = jnp.zeros((16,), jnp.int32)

        def fill(z, carry):
            oidx[pl.ds(z * 16, 16)] = (lanes + z * 16) & (N - 1)
            return carry
        lax.fori_loop(0, _QW * _ROW // 16, fill, 0)

        def per_group_set(g, carry):
            xs = [[x0v[pl.ds(g * 16 * _GRP + h * 16, 16)],
                   x1v[pl.ds(g * 16 * _GRP + h * 16, 16)],
                   x2v[pl.ds(g * 16 * _GRP + h * 16, 16)]]
                  for h in range(_GRP)]
            rows = [(g * 16 * _GRP + h * 16 + lanes) * _ROW
                    for h in range(_GRP)]

            @plsc.parallel_loop(0, N, step=_UNROLL,
                                carry=(zeros16,) * _GRP)
            def per_cand(jj, cursors):
                cur = list(cursors)
                for u in range(_UNROLL):
                    jsplat = zeros16 + (jj + u)
                    ys = [plsc.load_gather(y0v, [jsplat]),
                          plsc.load_gather(y1v, [jsplat]),
                          plsc.load_gather(y2v, [jsplat])]
                    for h in range(_GRP):
                        d = [ys[c] - xs[h][c] for c in range(3)]
                        dist = d[0] * d[0] + d[1] * d[1] + d[2] * d[2]
                        ok = dist <= RADIUS2
                        plsc.store_scatter(
                            oidx, [rows[h] + jnp.minimum(cur[h], _ROW - 1)],
                            jsplat, mask=ok)
                        cur[h] = cur[h] + ok.astype(jnp.int32)
                return tuple(cur)

            cur = per_cand
            for h in range(_GRP):
                ocnt[pl.ds(g * 16 * _GRP + h * 16, 16)] = (
                    jnp.minimum(cur[h], K_CAP))
            return carry

        lax.fori_loop(0, _NG // _GRP, per_group_set, 0)

        def compact(q, carry):
            for k in range(K_CAP // 16):
                cidx[pl.ds(q * K_CAP + k * 16, 16)] = (
                    oidx[pl.ds(q * _ROW + k * 16, 16)])
            return carry
        lax.fori_loop(0, _QW, compact, 0)

        pltpu.sync_copy(cidx, idxh.at[pl.ds(qbase * K_CAP, _QW * K_CAP)])
        pltpu.sync_copy(ocnt, cnth.at[pl.ds(qbase, _QW)])

    return search_k(x0, x1, x2, y0, y1, y2)


_BQ = 64


def _mlp_body(e_ref, b_ref, cnt_ref, w1_ref, b1_ref, w2_ref, b2_ref, o_ref):
    e = e_ref[...]
    ea = lax.slice(e, (0, 0), (_BQ * K_CAP, CH))
    ef = lax.slice(e, (0, CH), (_BQ * K_CAP, 2 * CH))
    t = ea.reshape(_BQ, K_CAP, CH) + b_ref[...][:, None, :]
    t = _gelu(t).reshape(_BQ * K_CAP, CH)
    t = _gelu(jnp.dot(t, w1_ref[...], preferred_element_type=jnp.float32)
              + b1_ref[...])
    v = jnp.dot(t, w2_ref[...], preferred_element_type=jnp.float32) + b2_ref[...]
    v = v * ef
    slot = lax.broadcasted_iota(jnp.int32, (_BQ, K_CAP), 1)
    mask = (slot < cnt_ref[...]).astype(jnp.float32)
    v = v.reshape(_BQ, K_CAP, CH) * mask[:, :, None]
    o_ref[...] = jnp.sum(v, axis=1)


def _mlp(E, B, cnt, W1, b1, W2, b2):
    grid = N // _BQ
    return pl.pallas_call(
        _mlp_body,
        grid=(grid,),
        in_specs=[
            pl.BlockSpec((_BQ * K_CAP, 2 * CH), lambda i: (i, 0)),
            pl.BlockSpec((_BQ, CH), lambda i: (i, 0)),
            pl.BlockSpec((_BQ, 1), lambda i: (i, 0)),
            pl.BlockSpec((CH, CH), lambda i: (0, 0)),
            pl.BlockSpec((1, CH), lambda i: (0, 0)),
            pl.BlockSpec((CH, CH), lambda i: (0, 0)),
            pl.BlockSpec((1, CH), lambda i: (0, 0)),
        ],
        out_specs=pl.BlockSpec((_BQ, CH), lambda i: (i, 0)),
        out_shape=jax.ShapeDtypeStruct((N, CH), jnp.float32),
    )(E, B, cnt.reshape(N, 1), W1, b1.reshape(1, CH), W2, b2.reshape(1, CH))


def kernel(y, x, f_y, W0, b0, W1, b1, W2, b2):
    idx_flat, cnt = _search(x[:, 0], x[:, 1], x[:, 2],
                            y[:, 0], y[:, 1], y[:, 2])
    G, B = _prep(y, x, f_y, W0, b0)
    E = _gather_edges(G, idx_flat)
    return _mlp(E, B, cnt, W1, b1, W2, b2)
